# baseline (speedup 1.0000x reference)
import functools

import jax
import jax.numpy as jnp
from jax.experimental import pallas as pl
from jax.experimental.pallas import tpu as pltpu

_B = 1024
_D = 1024
_N = 16384
_TEMP_INV = 20.0
_LMAX = 20.0
_SUB = 1024
_NSUB = 4
_NT = _SUB * _NSUB
_TILES = _N // _NT


def _loss_body(x_ref, t_ref, f_ref, o_ref, xn_ref, s_ref, tg_ref, *l_refs):
    i = pl.program_id(0)

    @pl.when(i == 0)
    def _init():
        x = x_ref[...]
        nrm = jnp.maximum(
            jnp.sqrt(jnp.sum(x * x, axis=1, keepdims=True)), 1e-12)
        xn_ref[...] = x / nrm
        s_ref[...] = jnp.zeros((_B, 1), jnp.float32)
        tg_ref[...] = jnp.zeros((_B, 1), jnp.float32)

    for j in range(_NSUB):
        l_refs[j][...] = jax.lax.dot_general(
            xn_ref[...], f_ref[j * _SUB:(j + 1) * _SUB, :],
            (((1,), (1,)), ((), ())),
            preferred_element_type=jnp.float32,
            precision=jax.lax.Precision.DEFAULT)

    s_acc = jnp.zeros((_B, 1), jnp.float32)
    tg_acc = jnp.zeros((_B, 1), jnp.float32)
    for j in range(_NSUB):
        l = l_refs[j][...] * _TEMP_INV
        s_acc += jnp.sum(jnp.exp(l - _LMAX), axis=1, keepdims=True)
        cols = (i * _NT + j * _SUB
                + jax.lax.broadcasted_iota(jnp.int32, (_B, _SUB), 1))
        hit = cols == t_ref[...]
        tg_acc += jnp.sum(jnp.where(hit, l, 0.0), axis=1, keepdims=True)
    s_ref[...] += s_acc
    tg_ref[...] += tg_acc

    @pl.when(i == _TILES - 1)
    def _fin():
        loss = _LMAX + jnp.log(s_ref[...]) - tg_ref[...]
        o_ref[...] = jnp.sum(loss, keepdims=True) * (1.0 / _B)


@functools.partial(jax.jit, static_argnames=())
def kernel(inputs, targets, features):
    out = pl.pallas_call(
        _loss_body,
        grid=(_TILES,),
        in_specs=[
            pl.BlockSpec((_B, _D), lambda i: (0, 0)),
            pl.BlockSpec((_B, 1), lambda i: (0, 0)),
            pl.BlockSpec((_NT, _D), lambda i: (i, 0)),
        ],
        out_specs=pl.BlockSpec((1, 1), lambda i: (0, 0)),
        out_shape=jax.ShapeDtypeStruct((1, 1), jnp.float32),
        scratch_shapes=[
            pltpu.VMEM((_B, _D), jnp.float32),
            pltpu.VMEM((_B, 1), jnp.float32),
            pltpu.VMEM((_B, 1), jnp.float32),
        ] + [pltpu.VMEM((_B, _SUB), jnp.float32) for _ in range(_NSUB)],
    )(inputs, targets.astype(jnp.int32).reshape(_B, 1), features)
    return out[0, 0]


# speedup vs baseline: 1.0054x; 1.0054x over previous
import functools

import jax
import jax.numpy as jnp
from jax.experimental import pallas as pl
from jax.experimental.pallas import tpu as pltpu

_B = 1024
_D = 1024
_N = 16384
_TEMP_INV = 20.0
_LMAX = 20.0
_SUB = 1024
_NSUB = 4
_NT = _SUB * _NSUB
_TILES = _N // _NT


def _loss_body(x_ref, t_ref, f_ref, o_ref, xn_ref, fbf_ref, s_ref, tg_ref,
               *l_refs):
    i = pl.program_id(0)

    @pl.when(i == 0)
    def _init():
        x = x_ref[...]
        nrm = jnp.maximum(
            jnp.sqrt(jnp.sum(x * x, axis=1, keepdims=True)), 1e-12)
        xn_ref[...] = (x / nrm).astype(jnp.bfloat16)
        s_ref[...] = jnp.zeros((_B, 1), jnp.float32)
        tg_ref[...] = jnp.zeros((_B, 1), jnp.float32)

    for j in range(_NSUB):
        sl = slice(j * _SUB, (j + 1) * _SUB)
        fbf_ref[sl, :] = f_ref[sl, :].astype(jnp.bfloat16)
        l_refs[j][...] = jax.lax.dot_general(
            xn_ref[...], fbf_ref[sl, :],
            (((1,), (1,)), ((), ())),
            preferred_element_type=jnp.float32)

    s_acc = jnp.zeros((_B, 1), jnp.float32)
    tg_acc = jnp.zeros((_B, 1), jnp.float32)
    for j in range(_NSUB):
        l = l_refs[j][...] * _TEMP_INV
        s_acc += jnp.sum(jnp.exp(l - _LMAX), axis=1, keepdims=True)
        cols = (i * _NT + j * _SUB
                + jax.lax.broadcasted_iota(jnp.int32, (_B, _SUB), 1))
        hit = cols == t_ref[...]
        tg_acc += jnp.sum(jnp.where(hit, l, 0.0), axis=1, keepdims=True)
    s_ref[...] += s_acc
    tg_ref[...] += tg_acc

    @pl.when(i == _TILES - 1)
    def _fin():
        loss = _LMAX + jnp.log(s_ref[...]) - tg_ref[...]
        o_ref[...] = jnp.sum(loss, keepdims=True) * (1.0 / _B)


@functools.partial(jax.jit, static_argnames=())
def kernel(inputs, targets, features):
    out = pl.pallas_call(
        _loss_body,
        grid=(_TILES,),
        in_specs=[
            pl.BlockSpec((_B, _D), lambda i: (0, 0)),
            pl.BlockSpec((_B, 1), lambda i: (0, 0)),
            pl.BlockSpec((_NT, _D), lambda i: (i, 0)),
        ],
        out_specs=pl.BlockSpec((1, 1), lambda i: (0, 0)),
        out_shape=jax.ShapeDtypeStruct((1, 1), jnp.float32),
        scratch_shapes=[
            pltpu.VMEM((_B, _D), jnp.bfloat16),
            pltpu.VMEM((_NT, _D), jnp.bfloat16),
            pltpu.VMEM((_B, 1), jnp.float32),
            pltpu.VMEM((_B, 1), jnp.float32),
        ] + [pltpu.VMEM((_B, _SUB), jnp.float32) for _ in range(_NSUB)],
    )(inputs, targets.astype(jnp.int32).reshape(_B, 1), features)
    return out[0, 0]


# speedup vs baseline: 1.0502x; 1.0446x over previous
import functools

import jax
import jax.numpy as jnp
from jax.experimental import pallas as pl
from jax.experimental.pallas import tpu as pltpu

_B = 1024
_D = 1024
_N = 16384
_TEMP_INV = 20.0
_LMAX = 20.0
_SUB = 1024
_NSUB = 2
_NT = _SUB * _NSUB
_TILES = _N // _NT


def _loss_body(x_ref, t_ref, f_ref, o_ref, xn_ref, s_ref, tg_ref, *l_refs):
    i = pl.program_id(0)

    @pl.when(i == 0)
    def _init():
        x = x_ref[...]
        nrm = jnp.maximum(
            jnp.sqrt(jnp.sum(x * x, axis=1, keepdims=True)), 1e-12)
        xn_ref[...] = x / nrm
        s_ref[...] = jnp.zeros((_B, 1), jnp.float32)
        tg_ref[...] = jnp.zeros((_B, 1), jnp.float32)

    for j in range(_NSUB):
        sl = slice(j * _SUB, (j + 1) * _SUB)
        l_refs[j][...] = jax.lax.dot_general(
            xn_ref[...], f_ref[sl, :],
            (((1,), (1,)), ((), ())),
            preferred_element_type=jnp.float32,
            precision=jax.lax.Precision.DEFAULT)

    s_acc = jnp.zeros((_B, 1), jnp.float32)
    tg_acc = jnp.zeros((_B, 1), jnp.float32)
    for j in range(_NSUB):
        l = l_refs[j][...] * _TEMP_INV
        s_acc += jnp.sum(jnp.exp(l - _LMAX), axis=1, keepdims=True)
        cols = (i * _NT + j * _SUB
                + jax.lax.broadcasted_iota(jnp.int32, (_B, _SUB), 1))
        hit = cols == t_ref[...]
        tg_acc += jnp.sum(jnp.where(hit, l, 0.0), axis=1, keepdims=True)
    s_ref[...] += s_acc
    tg_ref[...] += tg_acc

    @pl.when(i == _TILES - 1)
    def _fin():
        loss = _LMAX + jnp.log(s_ref[...]) - tg_ref[...]
        o_ref[...] = jnp.sum(loss, keepdims=True) * (1.0 / _B)


@functools.partial(jax.jit, static_argnames=())
def kernel(inputs, targets, features):
    out = pl.pallas_call(
        _loss_body,
        grid=(_TILES,),
        in_specs=[
            pl.BlockSpec((_B, _D), lambda i: (0, 0)),
            pl.BlockSpec((_B, 1), lambda i: (0, 0)),
            pl.BlockSpec((_NT, _D), lambda i: (i, 0)),
        ],
        out_specs=pl.BlockSpec((1, 1), lambda i: (0, 0)),
        out_shape=jax.ShapeDtypeStruct((1, 1), jnp.float32),
        scratch_shapes=[
            pltpu.VMEM((_B, _D), jnp.float32),
            pltpu.VMEM((_B, 1), jnp.float32),
            pltpu.VMEM((_B, 1), jnp.float32),
        ] + [pltpu.VMEM((_B, _SUB), jnp.float32) for _ in range(_NSUB)],
    )(inputs, targets.astype(jnp.int32).reshape(_B, 1), features)
    return out[0, 0]
